# baseline (device time: 8010 ns/iter reference)
import jax
import jax.numpy as jnp
from jax import lax
from jax.experimental import pallas as pl
from jax.experimental.pallas import tpu as pltpu

N_DEV = 4
TAPS = 4
HALO = TAPS - 1
NCHUNK = 4
PADOFF = 8


def kernel(x, k):
    b, s, c = x.shape
    cs = s // NCHUNK

    def body(x_hbm, k_ref, out_hbm, send_buf, halo_ref, pad_ref, out_vmem,
             stage_sem, send_sem, recv_sem, in_sems, out_sems):
        my = lax.axis_index("i")
        left = (my - 1) % N_DEV
        right = (my + 1) % N_DEV

        barrier_sem = pltpu.get_barrier_semaphore()
        pl.semaphore_signal(
            barrier_sem, inc=1,
            device_id=(left,), device_id_type=pl.DeviceIdType.MESH,
        )

        pad_ref[:, :PADOFF, :] = jnp.zeros((b, PADOFF, c), x_hbm.dtype)

        stage = pltpu.make_async_copy(
            x_hbm.at[:, pl.ds(s - HALO, HALO), :], send_buf, stage_sem
        )
        stage.start()
        in_copies = []
        for j in range(NCHUNK):
            cp = pltpu.make_async_copy(
                x_hbm.at[:, pl.ds(j * cs, cs), :],
                pad_ref.at[:, pl.ds(PADOFF + j * cs, cs), :],
                in_sems.at[j],
            )
            cp.start()
            in_copies.append(cp)

        stage.wait()
        pl.semaphore_wait(barrier_sem, 1)
        rdma = pltpu.make_async_remote_copy(
            src_ref=send_buf,
            dst_ref=halo_ref,
            send_sem=send_sem,
            recv_sem=recv_sem,
            device_id=(right,),
            device_id_type=pl.DeviceIdType.MESH,
        )
        rdma.start()

        kv = k_ref[...]

        out_copies = []
        for j in range(NCHUNK):
            in_copies[j].wait()
            base = j * cs
            acc = pad_ref[:, base + PADOFF - HALO:base + PADOFF - HALO + cs, :] * kv[0]
            for t in range(1, TAPS):
                o = base + PADOFF - HALO + t
                acc += pad_ref[:, o:o + cs, :] * kv[t]
            out_vmem[:, base:base + cs, :] = acc * jax.nn.sigmoid(acc)
            if j > 0:
                oc = pltpu.make_async_copy(
                    out_vmem.at[:, pl.ds(base, cs), :],
                    out_hbm.at[:, pl.ds(base, cs), :],
                    out_sems.at[j],
                )
                oc.start()
                out_copies.append(oc)

        rdma.wait()
        hv = halo_ref[...]
        pad_ref[:, PADOFF - HALO:PADOFF, :] = jnp.where(
            my == 0, jnp.zeros_like(hv), hv
        )
        head = pad_ref[:, PADOFF - HALO:PADOFF - HALO + HALO, :] * kv[0]
        for t in range(1, TAPS):
            o = PADOFF - HALO + t
            head += pad_ref[:, o:o + HALO, :] * kv[t]
        out_vmem[:, :HALO, :] = head * jax.nn.sigmoid(head)

        oc0 = pltpu.make_async_copy(
            out_vmem.at[:, pl.ds(0, cs), :],
            out_hbm.at[:, pl.ds(0, cs), :],
            out_sems.at[0],
        )
        oc0.start()
        oc0.wait()
        for oc in out_copies:
            oc.wait()

    return pl.pallas_call(
        body,
        out_shape=jax.ShapeDtypeStruct((b, s, c), x.dtype),
        in_specs=[
            pl.BlockSpec(memory_space=pl.ANY),
            pl.BlockSpec(memory_space=pltpu.VMEM),
        ],
        out_specs=pl.BlockSpec(memory_space=pl.ANY),
        scratch_shapes=[
            pltpu.VMEM((b, HALO, c), x.dtype),
            pltpu.VMEM((b, HALO, c), x.dtype),
            pltpu.VMEM((b, s + PADOFF, c), x.dtype),
            pltpu.VMEM((b, s, c), x.dtype),
            pltpu.SemaphoreType.DMA,
            pltpu.SemaphoreType.DMA,
            pltpu.SemaphoreType.DMA,
            pltpu.SemaphoreType.DMA((NCHUNK,)),
            pltpu.SemaphoreType.DMA((NCHUNK,)),
        ],
        compiler_params=pltpu.CompilerParams(collective_id=0),
    )(x, k)


# device time: 7754 ns/iter; 1.0330x vs baseline; 1.0330x over previous
import jax
import jax.numpy as jnp
from jax import lax
from jax.experimental import pallas as pl
from jax.experimental.pallas import tpu as pltpu

N_DEV = 4
TAPS = 4
HALO = TAPS - 1
NCHUNK = 2
PADOFF = 8


def kernel(x, k):
    b, s, c = x.shape
    cs = s // NCHUNK

    def body(x_hbm, k_ref, out_hbm, send_buf, halo_ref, pad_ref, out_vmem,
             stage_sem, send_sem, recv_sem, in_sems, out_sems):
        my = lax.axis_index("i")
        left = (my - 1) % N_DEV
        right = (my + 1) % N_DEV

        barrier_sem = pltpu.get_barrier_semaphore()
        pl.semaphore_signal(
            barrier_sem, inc=1,
            device_id=(left,), device_id_type=pl.DeviceIdType.MESH,
        )

        pad_ref[:, :PADOFF, :] = jnp.zeros((b, PADOFF, c), x_hbm.dtype)

        stage = pltpu.make_async_copy(
            x_hbm.at[:, pl.ds(s - HALO, HALO), :], send_buf, stage_sem
        )
        stage.start()
        in_copies = []
        for j in range(NCHUNK):
            cp = pltpu.make_async_copy(
                x_hbm.at[:, pl.ds(j * cs, cs), :],
                pad_ref.at[:, pl.ds(PADOFF + j * cs, cs), :],
                in_sems.at[j],
            )
            cp.start()
            in_copies.append(cp)

        stage.wait()
        pl.semaphore_wait(barrier_sem, 1)
        rdma = pltpu.make_async_remote_copy(
            src_ref=send_buf,
            dst_ref=halo_ref,
            send_sem=send_sem,
            recv_sem=recv_sem,
            device_id=(right,),
            device_id_type=pl.DeviceIdType.MESH,
        )
        rdma.start()

        kv = k_ref[...]

        out_copies = []
        for j in range(NCHUNK):
            in_copies[j].wait()
            base = j * cs
            acc = pad_ref[:, base + PADOFF - HALO:base + PADOFF - HALO + cs, :] * kv[0]
            for t in range(1, TAPS):
                o = base + PADOFF - HALO + t
                acc += pad_ref[:, o:o + cs, :] * kv[t]
            out_vmem[:, base:base + cs, :] = acc * jax.nn.sigmoid(acc)
            if j > 0:
                oc = pltpu.make_async_copy(
                    out_vmem.at[:, pl.ds(base, cs), :],
                    out_hbm.at[:, pl.ds(base, cs), :],
                    out_sems.at[j],
                )
                oc.start()
                out_copies.append(oc)

        rdma.wait()
        hv = halo_ref[...]
        pad_ref[:, PADOFF - HALO:PADOFF, :] = jnp.where(
            my == 0, jnp.zeros_like(hv), hv
        )
        head = pad_ref[:, PADOFF - HALO:PADOFF - HALO + HALO, :] * kv[0]
        for t in range(1, TAPS):
            o = PADOFF - HALO + t
            head += pad_ref[:, o:o + HALO, :] * kv[t]
        out_vmem[:, :HALO, :] = head * jax.nn.sigmoid(head)

        oc0 = pltpu.make_async_copy(
            out_vmem.at[:, pl.ds(0, cs), :],
            out_hbm.at[:, pl.ds(0, cs), :],
            out_sems.at[0],
        )
        oc0.start()
        oc0.wait()
        for oc in out_copies:
            oc.wait()

    return pl.pallas_call(
        body,
        out_shape=jax.ShapeDtypeStruct((b, s, c), x.dtype),
        in_specs=[
            pl.BlockSpec(memory_space=pl.ANY),
            pl.BlockSpec(memory_space=pltpu.VMEM),
        ],
        out_specs=pl.BlockSpec(memory_space=pl.ANY),
        scratch_shapes=[
            pltpu.VMEM((b, HALO, c), x.dtype),
            pltpu.VMEM((b, HALO, c), x.dtype),
            pltpu.VMEM((b, s + PADOFF, c), x.dtype),
            pltpu.VMEM((b, s, c), x.dtype),
            pltpu.SemaphoreType.DMA,
            pltpu.SemaphoreType.DMA,
            pltpu.SemaphoreType.DMA,
            pltpu.SemaphoreType.DMA((NCHUNK,)),
            pltpu.SemaphoreType.DMA((NCHUNK,)),
        ],
        compiler_params=pltpu.CompilerParams(collective_id=0),
    )(x, k)


# device time: 7678 ns/iter; 1.0432x vs baseline; 1.0099x over previous
import jax
import jax.numpy as jnp
from jax import lax
from jax.experimental import pallas as pl
from jax.experimental.pallas import tpu as pltpu

N_DEV = 4
TAPS = 4
HALO = TAPS - 1


def kernel(x, k):
    b, s, c = x.shape

    def body(x_ref, k_ref, out_ref, send_buf, halo_ref, pad_ref, send_sem, recv_sem):
        my = lax.axis_index("i")
        left = (my - 1) % N_DEV
        right = (my + 1) % N_DEV

        barrier_sem = pltpu.get_barrier_semaphore()
        pl.semaphore_signal(
            barrier_sem, inc=1,
            device_id=(left,), device_id_type=pl.DeviceIdType.MESH,
        )

        send_buf[...] = x_ref[:, s - HALO:, :]
        rdma = pltpu.make_async_remote_copy(
            src_ref=send_buf,
            dst_ref=halo_ref,
            send_sem=send_sem,
            recv_sem=recv_sem,
            device_id=(right,),
            device_id_type=pl.DeviceIdType.MESH,
        )

        pl.semaphore_wait(barrier_sem, 1)
        rdma.start()

        pad_ref[:, :HALO, :] = jnp.zeros((b, HALO, c), x_ref.dtype)
        pad_ref[:, HALO:, :] = x_ref[...]

        kv = k_ref[...]
        acc = pad_ref[:, 0:s, :] * kv[0]
        for t in range(1, TAPS):
            acc += pad_ref[:, t:t + s, :] * kv[t]
        out_ref[...] = acc * jax.nn.sigmoid(acc)

        rdma.wait()
        hv = halo_ref[...]
        hv = jnp.where(my == 0, jnp.zeros_like(hv), hv)
        rows = []
        for j in range(HALO):
            r = hv[:, j:j + 1, :] * kv[0]
            for t in range(1, HALO - j):
                r += hv[:, j + t:j + t + 1, :] * kv[t]
            rows.append(r)
        corr = jnp.concatenate(rows, axis=1)
        head = acc[:, :HALO, :] + corr
        out_ref[:, :HALO, :] = head * jax.nn.sigmoid(head)

    return pl.pallas_call(
        body,
        out_shape=jax.ShapeDtypeStruct((b, s, c), x.dtype),
        in_specs=[
            pl.BlockSpec(memory_space=pltpu.VMEM),
            pl.BlockSpec(memory_space=pltpu.VMEM),
        ],
        out_specs=pl.BlockSpec(memory_space=pltpu.VMEM),
        scratch_shapes=[
            pltpu.VMEM((b, HALO, c), x.dtype),
            pltpu.VMEM((b, HALO, c), x.dtype),
            pltpu.VMEM((b, s + HALO, c), x.dtype),
            pltpu.SemaphoreType.DMA,
            pltpu.SemaphoreType.DMA,
        ],
        compiler_params=pltpu.CompilerParams(collective_id=0),
    )(x, k)
